# baseline (speedup 1.0000x reference)
import jax
import jax.numpy as jnp
from jax.experimental import pallas as pl
from jax.experimental.pallas import tpu as pltpu

H = 8
S = 256
T = 1024
D = 64
SZ_B = 16
B = SZ_B * H
RATIO = T // S
W = 2 * S

T_BLK = 512
G = T_BLK // RATIO
WIN = S + G
W_PAD = (S - G) + WIN


def _rel_attn_kernel(q_ref, c_ref, idxw_ref, cross_ref, o_ref):
    j = pl.program_id(2)
    base = (S - G) - G * j
    R_SUB = 64
    dims = (((1,), (1,)), ((), ()))
    for k in range(T_BLK // R_SUB):
        qk = q_ref[0, pl.ds(k * R_SUB, R_SUB), :]
        tiles = [
            jax.lax.dot_general(
                qk, c_ref[0, pl.ds(base + c * 128, 128), :], dims,
                preferred_element_type=jnp.float32)
            for c in range(WIN // 128)
        ]
        idxw = idxw_ref[0, pl.ds(k * R_SUB, R_SUB), :]
        cross = cross_ref[0, pl.ds(k * R_SUB, R_SUB), :] != 0
        for c in range(S // 128):
            g_a = jnp.take_along_axis(tiles[c], idxw, axis=1)
            g_b = jnp.take_along_axis(tiles[c + 1], idxw, axis=1)
            o_ref[0, pl.ds(k * R_SUB, R_SUB), pl.ds(c * 128, 128)] = (
                jnp.where(cross, g_b, g_a))


@jax.jit
def kernel(q, e1, e2):
    e1h = e1.reshape(H, S, D)
    e2h = e2.reshape(H, S, D)
    c = jnp.concatenate(
        [e1h, e2h[:, 1:, :], jnp.zeros((H, 1 + W_PAD - W, D), e2h.dtype)],
        axis=1)

    resid = (G - 1) - (jnp.arange(T, dtype=jnp.int32)[:, None] % T_BLK) // RATIO
    idx = jnp.arange(128, dtype=jnp.int32)[None, :] + resid
    idxw = (idx & 127).reshape(T // T_BLK, T_BLK, 128)
    cross = (idx >= 128).astype(jnp.int32).reshape(T // T_BLK, T_BLK, 128)

    grid = (H, SZ_B, T // T_BLK)
    return pl.pallas_call(
        _rel_attn_kernel,
        grid=grid,
        in_specs=[
            pl.BlockSpec((1, T_BLK, D), lambda h, b, j: (b * H + h, j, 0)),
            pl.BlockSpec((1, W_PAD, D), lambda h, b, j: (h, 0, 0)),
            pl.BlockSpec((1, T_BLK, 128), lambda h, b, j: (j, 0, 0)),
            pl.BlockSpec((1, T_BLK, 128), lambda h, b, j: (j, 0, 0)),
        ],
        out_specs=pl.BlockSpec((1, T_BLK, S), lambda h, b, j: (b * H + h, j, 0)),
        out_shape=jax.ShapeDtypeStruct((B, T, S), jnp.float32),
        compiler_params=pltpu.CompilerParams(
            dimension_semantics=("parallel", "parallel", "arbitrary"),
        ),
    )(q, c, idxw, cross)


# speedup vs baseline: 1.1521x; 1.1521x over previous
import jax
import jax.numpy as jnp
from jax.experimental import pallas as pl
from jax.experimental.pallas import tpu as pltpu

H = 8
S = 256
T = 1024
D = 64
SZ_B = 16
B = SZ_B * H
RATIO = T // S
W = 2 * S

T_BLK = 512
G = T_BLK // RATIO
WIN = S + G
W_PAD = (S - G) + WIN


def _rel_attn_kernel(q_ref, c_ref, o_ref):
    j = pl.program_id(2)
    base = (S - G) - G * j
    c_win = c_ref[0, pl.ds(base, WIN), :]
    R_SUB = 64
    s128 = jax.lax.broadcasted_iota(jnp.int32, (R_SUB, 128), 1)
    r_sub = jax.lax.broadcasted_iota(jnp.int32, (R_SUB, 1), 0)
    for k in range(T_BLK // R_SUB):
        sc = jax.lax.dot_general(
            q_ref[0, pl.ds(k * R_SUB, R_SUB), :], c_win,
            (((1,), (1,)), ((), ())),
            preferred_element_type=jnp.float32,
        )
        resid = (G - 1) - (k * R_SUB + r_sub) // RATIO
        idx = s128 + resid
        idxw = idx & 127
        cross = idx >= 128
        for c in range(S // 128):
            src_a = sc[:, c * 128:(c + 1) * 128]
            src_b = sc[:, (c + 1) * 128:(c + 2) * 128]
            g_a = jnp.take_along_axis(src_a, idxw, axis=1)
            g_b = jnp.take_along_axis(src_b, idxw, axis=1)
            o_ref[0, pl.ds(k * R_SUB, R_SUB), pl.ds(c * 128, 128)] = (
                jnp.where(cross, g_b, g_a))


@jax.jit
def kernel(q, e1, e2):
    e1h = e1.reshape(H, S, D)
    e2h = e2.reshape(H, S, D)
    c = jnp.concatenate(
        [e1h, e2h[:, 1:, :], jnp.zeros((H, 1 + W_PAD - W, D), e2h.dtype)],
        axis=1)

    grid = (H, SZ_B, T // T_BLK)
    return pl.pallas_call(
        _rel_attn_kernel,
        grid=grid,
        in_specs=[
            pl.BlockSpec((1, T_BLK, D), lambda h, b, j: (b * H + h, j, 0)),
            pl.BlockSpec((1, W_PAD, D), lambda h, b, j: (h, 0, 0)),
        ],
        out_specs=pl.BlockSpec((1, T_BLK, S), lambda h, b, j: (b * H + h, j, 0)),
        out_shape=jax.ShapeDtypeStruct((B, T, S), jnp.float32),
        compiler_params=pltpu.CompilerParams(
            dimension_semantics=("parallel", "parallel", "arbitrary"),
        ),
    )(q, c)
